# baseline (speedup 1.0000x reference)
import jax
import jax.numpy as jnp
from jax import lax
from jax.experimental import pallas as pl
from jax.experimental.pallas import tpu as pltpu
from jax.experimental.pallas import tpu_sc as plsc

_CYCLE = 168
_OUT_LEN = 720
_CHAN = 128
_BATCH = 1024
_EXT = 888
_NW = 32
_BPW = _BATCH // _NW


def _sc_body(start_hbm, data_hbm, out_hbm, idx_v, table_v, stab_sh, sem):
    sid = lax.axis_index("s")
    wid = sid * 2 + lax.axis_index("c")
    base = wid * _BPW

    stage = [pltpu.make_async_copy(start_hbm.at[pl.ds(base, _BPW)], idx_v, sem)]
    nfull = _EXT // _CYCLE
    for k in range(nfull):
        stage.append(pltpu.make_async_copy(
            data_hbm, table_v.at[pl.ds(k * _CYCLE, _CYCLE)], sem))
    rem = _EXT - nfull * _CYCLE
    stage.append(pltpu.make_async_copy(
        data_hbm.at[pl.ds(0, rem)], table_v.at[pl.ds(nfull * _CYCLE, rem)], sem))
    for cp in stage:
        cp.start()

    @pl.when(sid == 0)
    def _build_shared():
        for k in range(nfull):
            pltpu.sync_copy(data_hbm, stab_sh.at[pl.ds(k * _CYCLE, _CYCLE)])
        pltpu.sync_copy(data_hbm.at[pl.ds(0, rem)],
                        stab_sh.at[pl.ds(nfull * _CYCLE, rem)])

    for cp in stage:
        cp.wait()
    plsc.subcore_barrier()

    copies = []
    for c in range(_BPW // 16):
        vec = idx_v[pl.ds(c * 16, 16)]
        for ln in range(16):
            s = vec[ln]
            src_ref = table_v if (ln % 2 == 0) else stab_sh
            cp = pltpu.make_async_copy(
                src_ref.at[pl.ds(s, _OUT_LEN)],
                out_hbm.at[base + c * 16 + ln],
                sem,
            )
            cp.start()
            copies.append(cp)
    for cp in copies:
        cp.wait()


def kernel(cycle_index, output_len, data):
    delta = jnp.asarray(output_len, jnp.int32) - _OUT_LEN
    start = jnp.mod(cycle_index.astype(jnp.int32) + delta, _CYCLE)

    run = pl.kernel(
        _sc_body,
        out_type=jax.ShapeDtypeStruct((_BATCH, _OUT_LEN, _CHAN), jnp.float32),
        mesh=plsc.VectorSubcoreMesh(core_axis_name="c", subcore_axis_name="s"),
        scratch_types=[
            pltpu.VMEM((_BPW,), jnp.int32),
            pltpu.VMEM((_EXT, _CHAN), jnp.float32),
            pltpu.VMEM_SHARED((_EXT, _CHAN), jnp.float32),
            pltpu.SemaphoreType.DMA,
        ],
    )
    return run(start, data)


# speedup vs baseline: 1.0397x; 1.0397x over previous
import jax
import jax.numpy as jnp
from jax import lax
from jax.experimental import pallas as pl
from jax.experimental.pallas import tpu as pltpu
from jax.experimental.pallas import tpu_sc as plsc

_CYCLE = 168
_OUT_LEN = 720
_CHAN = 128
_BATCH = 1024
_EXT = 888
_NW = 32
_BPW = _BATCH // _NW


def _sc_body(start_hbm, data_hbm, out_hbm, idx_v, table_v, sem):
    wid = lax.axis_index("s") * 2 + lax.axis_index("c")
    base = wid * _BPW

    stage = [pltpu.make_async_copy(start_hbm.at[pl.ds(base, _BPW)], idx_v, sem)]
    nfull = _EXT // _CYCLE
    for k in range(nfull):
        stage.append(pltpu.make_async_copy(
            data_hbm, table_v.at[pl.ds(k * _CYCLE, _CYCLE)], sem))
    rem = _EXT - nfull * _CYCLE
    stage.append(pltpu.make_async_copy(
        data_hbm.at[pl.ds(0, rem)], table_v.at[pl.ds(nfull * _CYCLE, rem)], sem))
    for cp in stage:
        cp.start()
    for cp in stage:
        cp.wait()

    copies = []
    for c in range(_BPW // 16):
        vec = idx_v[pl.ds(c * 16, 16)]
        for ln in range(16):
            s = vec[ln]
            cp = pltpu.make_async_copy(
                table_v.at[pl.ds(s, _OUT_LEN)],
                out_hbm.at[base + c * 16 + ln],
                sem,
            )
            cp.start()
            copies.append(cp)
    for cp in copies:
        cp.wait()


def kernel(cycle_index, output_len, data):
    delta = jnp.asarray(output_len, jnp.int32) - _OUT_LEN
    start = jnp.mod(cycle_index.astype(jnp.int32) + delta, _CYCLE)

    run = pl.kernel(
        _sc_body,
        out_type=jax.ShapeDtypeStruct((_BATCH, _OUT_LEN, _CHAN), jnp.float32),
        mesh=plsc.VectorSubcoreMesh(core_axis_name="c", subcore_axis_name="s"),
        scratch_types=[
            pltpu.VMEM((_BPW,), jnp.int32),
            pltpu.VMEM((_EXT, _CHAN), jnp.float32),
            pltpu.SemaphoreType.DMA,
        ],
    )
    return run(start, data)
